# Initial kernel scaffold; baseline (speedup 1.0000x reference)
import functools

import jax
import jax.numpy as jnp
from jax import lax
from jax.experimental import pallas as pl
from jax.experimental.pallas import tpu as pltpu
from jax.experimental.pallas import tpu_sc as plsc

N = 100000
D = 128
OUT = 128
S = 10
B = 4096

NC = 2
NS = 16
NW = NC * NS
CH = B // NW
LN = 16

BB = 256
NBLK = B // BB


def _sc_gather(raw, nidx16, nodes):
    f32, i32 = jnp.float32, jnp.int32
    mesh = plsc.VectorSubcoreMesh(core_axis_name="c", subcore_axis_name="s")
    out_type = [
        jax.ShapeDtypeStruct((B, D), f32),
        jax.ShapeDtypeStruct((B, D), f32),
        jax.ShapeDtypeStruct((B * S, D), f32),
        jax.ShapeDtypeStruct((B * S, D), f32),
    ]
    scratch = [
        pltpu.VMEM((CH,), i32),
        pltpu.VMEM((CH, 16), i32),
        pltpu.VMEM((CH,), i32),
        pltpu.VMEM((CH, 16), i32),
        pltpu.VMEM((CH,), i32),
        pltpu.VMEM((CH, D), f32),
        pltpu.VMEM((CH, D), f32),
        pltpu.SemaphoreType.DMA,
    ]

    @functools.partial(pl.kernel, mesh=mesh, out_type=out_type,
                       scratch_types=scratch)
    def k(raw_h, nidx_h, nodes_h, self_b, total_b, self_n, total_n,
          nbv, nrows, cur, n1, slot, acc, buf, sem):
        wid = lax.axis_index("s") * NC + lax.axis_index("c")
        base = wid * CH

        pltpu.sync_copy(nodes_h.at[pl.ds(base, CH)], nbv)
        pltpu.async_copy(nidx_h.at[nbv], nrows, sem).wait()

        def extract_col(src, col, dst):
            colv = jnp.full((LN,), col, i32)
            for kk in range(CH // LN):
                rows = lax.iota(i32, LN) + LN * kk
                dst[pl.ds(LN * kk, LN)] = plsc.load_gather(src, [rows, colv])

        def accumulate():
            def body(r, carry):
                for kk in range(D // LN):
                    sl = pl.ds(kk * LN, LN)
                    plsc.addupdate(acc.at[r, sl], buf[r, sl])
                return carry
            lax.fori_loop(0, CH, body, 0)

        def process(idx_ref, self_out, total_out, row0):
            pltpu.async_copy(nidx_h.at[idx_ref], n1, sem).wait()
            pltpu.async_copy(raw_h.at[idx_ref], acc, sem).wait()
            pltpu.sync_copy(acc, self_out.at[pl.ds(row0, CH)])
            for jj in range(S):
                extract_col(n1, jj, slot)
                pltpu.async_copy(raw_h.at[slot], buf, sem).wait()
                accumulate()
            pltpu.sync_copy(acc, total_out.at[pl.ds(row0, CH)])

        process(nbv, self_b, total_b, base)

        def nbody(j, carry):
            extract_col(nrows, j, cur)
            process(cur, self_n, total_n, j * B + base)
            return carry
        lax.fori_loop(0, S, nbody, 0)

    return k(raw, nidx16, nodes)


def _tc_fused(sb, tb, sn, tn, wsa, wsb, w2a, w2b):
    f32 = jnp.float32

    def body(sb_r, tb_r, sn_r, tn_r, wsa_r, wsb_r, w2a_r, w2b_r, out_r,
             h1b_s, acc_s):
        j = pl.program_id(1)
        h1n = jnp.maximum(
            jnp.dot(sn_r[:], wsa_r[:], preferred_element_type=f32)
            + jnp.dot(tn_r[:], wsb_r[:], preferred_element_type=f32), 0.0)

        @pl.when(j == 0)
        def _():
            h1b_s[:] = jnp.maximum(
                jnp.dot(sb_r[:], wsa_r[:], preferred_element_type=f32)
                + jnp.dot(tb_r[:], wsb_r[:], preferred_element_type=f32), 0.0)
            acc_s[:] = h1n

        @pl.when(j > 0)
        def _():
            acc_s[:] = acc_s[:] + h1n

        @pl.when(j == S - 1)
        def _():
            h1b = h1b_s[:]
            out_r[:] = jnp.maximum(
                jnp.dot(h1b, w2a_r[:], preferred_element_type=f32)
                + jnp.dot(acc_s[:] + h1b, w2b_r[:], preferred_element_type=f32),
                0.0)

    return pl.pallas_call(
        body,
        grid=(NBLK, S),
        in_specs=[
            pl.BlockSpec((BB, D), lambda ib, j: (ib, 0)),
            pl.BlockSpec((BB, D), lambda ib, j: (ib, 0)),
            pl.BlockSpec((BB, D), lambda ib, j: (j * NBLK + ib, 0)),
            pl.BlockSpec((BB, D), lambda ib, j: (j * NBLK + ib, 0)),
            pl.BlockSpec((D, OUT), lambda ib, j: (0, 0)),
            pl.BlockSpec((D, OUT), lambda ib, j: (0, 0)),
            pl.BlockSpec((OUT, OUT), lambda ib, j: (0, 0)),
            pl.BlockSpec((OUT, OUT), lambda ib, j: (0, 0)),
        ],
        out_specs=pl.BlockSpec((BB, OUT), lambda ib, j: (ib, 0)),
        out_shape=jax.ShapeDtypeStruct((B, OUT), jnp.float32),
        scratch_shapes=[pltpu.VMEM((BB, OUT), jnp.float32),
                        pltpu.VMEM((BB, OUT), jnp.float32)],
        compiler_params=pltpu.CompilerParams(
            dimension_semantics=("arbitrary", "arbitrary")),
    )(sb, tb, sn, tn, wsa, wsb, w2a, w2b)


def kernel(raw_features, neigh_idx, nodes_batch, W1, W2):
    nidx16 = jnp.concatenate(
        [neigh_idx.astype(jnp.int32),
         jnp.zeros((N, 16 - S), jnp.int32)], axis=1)
    nodes = nodes_batch.astype(jnp.int32)

    self_b, total_b, self_n, total_n = _sc_gather(raw_features, nidx16, nodes)

    inv = 1.0 / (S + 1)
    wsa = W1[:, :D].T
    wsb = W1[:, D:].T * inv
    w2a = W2[:, :OUT].T
    w2b = W2[:, OUT:].T * inv
    return _tc_fused(self_b, total_b, self_n, total_n, wsa, wsb, w2a, w2b)


# speedup vs baseline: 4.3360x; 4.3360x over previous
import functools

import jax
import jax.numpy as jnp
from jax import lax
from jax.experimental import pallas as pl
from jax.experimental.pallas import tpu as pltpu
from jax.experimental.pallas import tpu_sc as plsc

N = 100000
D = 128
OUT = 128
S = 10
B = 4096

NC = 2
NS = 16
NW = NC * NS
CH = B // NW
LN = 16

BB = 256
NBLK = B // BB


def _sc_gather(raw, nidxT, nodes):
    f32, i32 = jnp.float32, jnp.int32
    mesh = plsc.VectorSubcoreMesh(core_axis_name="c", subcore_axis_name="s")
    out_type = [
        jax.ShapeDtypeStruct((B, D), f32),
        jax.ShapeDtypeStruct((B, D), f32),
        jax.ShapeDtypeStruct((B * S, D), f32),
        jax.ShapeDtypeStruct((B * S, D), f32),
    ]
    scratch = [
        pltpu.VMEM((CH,), i32),
        pltpu.VMEM((CH,), i32),
        pltpu.VMEM((CH,), i32),
        pltpu.VMEM((CH,), i32),
        pltpu.VMEM((CH, D), f32),
        pltpu.VMEM((CH, D), f32),
        pltpu.SemaphoreType.DMA,
    ]

    @functools.partial(pl.kernel, mesh=mesh, out_type=out_type,
                       scratch_types=scratch)
    def k(raw_h, nt_h, nodes_h, self_b, total_b, self_n, total_n,
          nbv, cur, idxb, slot, acc, buf, sem):
        wid = lax.axis_index("s") * NC + lax.axis_index("c")
        base = wid * CH

        pltpu.sync_copy(nodes_h.at[pl.ds(base, CH)], nbv)

        def addv(src_ref, off, dst_ref):
            offv = jnp.full((LN,), off, i32)
            for kk in range(CH // LN):
                sl = pl.ds(kk * LN, LN)
                dst_ref[sl] = src_ref[sl] + offv

        def accumulate():
            def body(r, carry):
                for kk in range(D // LN):
                    sl = pl.ds(kk * LN, LN)
                    plsc.addupdate(acc.at[r, sl], buf[r, sl])
                return carry
            lax.fori_loop(0, CH, body, 0)

        def process(node_ref, self_out, total_out, row0):
            pltpu.async_copy(raw_h.at[node_ref], acc, sem).wait()
            pltpu.sync_copy(acc, self_out.at[pl.ds(row0, CH)])

            def sbody(s, carry):
                addv(node_ref, s * N, idxb)
                pltpu.async_copy(nt_h.at[idxb], slot, sem).wait()
                pltpu.async_copy(raw_h.at[slot], buf, sem).wait()
                accumulate()
                return carry
            lax.fori_loop(0, S, sbody, 0)
            pltpu.sync_copy(acc, total_out.at[pl.ds(row0, CH)])

        process(nbv, self_b, total_b, base)

        def nbody(j, carry):
            addv(nbv, j * N, idxb)
            pltpu.async_copy(nt_h.at[idxb], cur, sem).wait()
            process(cur, self_n, total_n, j * B + base)
            return carry
        lax.fori_loop(0, S, nbody, 0)

    return k(raw, nidxT, nodes)


def _tc_fused(sb, tb, sn, tn, wsa, wsb, w2a, w2b):
    f32 = jnp.float32

    def body(sb_r, tb_r, sn_r, tn_r, wsa_r, wsb_r, w2a_r, w2b_r, out_r,
             h1b_s, acc_s):
        j = pl.program_id(1)
        h1n = jnp.maximum(
            jnp.dot(sn_r[:], wsa_r[:], preferred_element_type=f32)
            + jnp.dot(tn_r[:], wsb_r[:], preferred_element_type=f32), 0.0)

        @pl.when(j == 0)
        def _():
            h1b_s[:] = jnp.maximum(
                jnp.dot(sb_r[:], wsa_r[:], preferred_element_type=f32)
                + jnp.dot(tb_r[:], wsb_r[:], preferred_element_type=f32), 0.0)
            acc_s[:] = h1n

        @pl.when(j > 0)
        def _():
            acc_s[:] = acc_s[:] + h1n

        @pl.when(j == S - 1)
        def _():
            h1b = h1b_s[:]
            out_r[:] = jnp.maximum(
                jnp.dot(h1b, w2a_r[:], preferred_element_type=f32)
                + jnp.dot(acc_s[:] + h1b, w2b_r[:], preferred_element_type=f32),
                0.0)

    return pl.pallas_call(
        body,
        grid=(NBLK, S),
        in_specs=[
            pl.BlockSpec((BB, D), lambda ib, j: (ib, 0)),
            pl.BlockSpec((BB, D), lambda ib, j: (ib, 0)),
            pl.BlockSpec((BB, D), lambda ib, j: (j * NBLK + ib, 0)),
            pl.BlockSpec((BB, D), lambda ib, j: (j * NBLK + ib, 0)),
            pl.BlockSpec((D, OUT), lambda ib, j: (0, 0)),
            pl.BlockSpec((D, OUT), lambda ib, j: (0, 0)),
            pl.BlockSpec((OUT, OUT), lambda ib, j: (0, 0)),
            pl.BlockSpec((OUT, OUT), lambda ib, j: (0, 0)),
        ],
        out_specs=pl.BlockSpec((BB, OUT), lambda ib, j: (ib, 0)),
        out_shape=jax.ShapeDtypeStruct((B, OUT), jnp.float32),
        scratch_shapes=[pltpu.VMEM((BB, OUT), jnp.float32),
                        pltpu.VMEM((BB, OUT), jnp.float32)],
        compiler_params=pltpu.CompilerParams(
            dimension_semantics=("arbitrary", "arbitrary")),
    )(sb, tb, sn, tn, wsa, wsb, w2a, w2b)


def kernel(raw_features, neigh_idx, nodes_batch, W1, W2):
    nidxT = neigh_idx.astype(jnp.int32).T.reshape(-1)
    nodes = nodes_batch.astype(jnp.int32)

    self_b, total_b, self_n, total_n = _sc_gather(raw_features, nidxT, nodes)

    inv = 1.0 / (S + 1)
    wsa = W1[:, :D].T
    wsb = W1[:, D:].T * inv
    w2a = W2[:, :OUT].T
    w2b = W2[:, OUT:].T * inv
    return _tc_fused(self_b, total_b, self_n, total_n, wsa, wsb, w2a, w2b)


# speedup vs baseline: 6.8643x; 1.5831x over previous
import functools

import jax
import jax.numpy as jnp
from jax import lax
from jax.experimental import pallas as pl
from jax.experimental.pallas import tpu as pltpu
from jax.experimental.pallas import tpu_sc as plsc

N = 100000
D = 128
OUT = 128
S = 10
B = 4096

NC = 2
NS = 16
NW = NC * NS
CH = B // NW
LN = 16

BB = 256
NBLK = B // BB


def _sc_gather(raw, nidxT, nodes):
    f32, i32 = jnp.float32, jnp.int32
    mesh = plsc.VectorSubcoreMesh(core_axis_name="c", subcore_axis_name="s")
    out_type = [
        jax.ShapeDtypeStruct((B, D), f32),
        jax.ShapeDtypeStruct((B, D), f32),
        jax.ShapeDtypeStruct((B * S, D), f32),
        jax.ShapeDtypeStruct((B * S, D), f32),
    ]
    scratch = [
        pltpu.VMEM((CH,), i32),
        pltpu.VMEM((S, CH), i32),
        pltpu.VMEM((S, CH), i32),
        pltpu.VMEM((S, CH), i32),
        pltpu.VMEM((CH, D), f32),
        pltpu.VMEM((CH, D), f32),
        pltpu.VMEM((CH, D), f32),
        pltpu.SemaphoreType.DMA,
        pltpu.SemaphoreType.DMA,
        pltpu.SemaphoreType.DMA,
        pltpu.SemaphoreType.DMA,
    ]

    @functools.partial(pl.kernel, mesh=mesh, out_type=out_type,
                       scratch_types=scratch)
    def k(raw_h, nt_h, nodes_h, self_b, total_b, self_n, total_n,
          nbv, idxs, curall, slots, acc, bufA, bufB,
          sem_self, sem_idx, semA, semB):
        wid = lax.axis_index("s") * NC + lax.axis_index("c")
        base = wid * CH

        pltpu.sync_copy(nodes_h.at[pl.ds(base, CH)], nbv)

        def addv(src_ref, off, dst_row):
            offv = jnp.full((LN,), off, i32)
            for kk in range(CH // LN):
                sl = pl.ds(kk * LN, LN)
                idxs[dst_row, sl] = src_ref[sl] + offv

        def fire_idx_gathers(node_ref, dst):
            def fire(s, carry):
                addv(node_ref, s * N, s)
                pltpu.async_copy(nt_h.at[idxs.at[s]], dst.at[s], sem_idx)
                return carry
            lax.fori_loop(0, S, fire, 0)

            def drain(s, carry):
                pltpu.make_async_copy(nt_h.at[idxs.at[s]], dst.at[s],
                                      sem_idx).wait()
                return carry
            lax.fori_loop(0, S, drain, 0)

        def accumulate(buf):
            def body(r2, carry):
                for dr in range(2):
                    r = r2 * 2 + dr
                    for kk in range(D // LN):
                        sl = pl.ds(kk * LN, LN)
                        plsc.addupdate(acc.at[r, sl], buf[r, sl])
                return carry
            lax.fori_loop(0, CH // 2, body, 0)

        def process(node_ref, self_out, total_out, row0):
            cp_self = pltpu.async_copy(raw_h.at[node_ref], acc, sem_self)
            fire_idx_gathers(node_ref, slots)
            cp_self.wait()
            pltpu.sync_copy(acc, self_out.at[pl.ds(row0, CH)])

            cps = {0: pltpu.async_copy(raw_h.at[slots.at[0]], bufA, semA)}
            for s in range(S):
                buf = bufA if s % 2 == 0 else bufB
                if s + 1 < S:
                    nxt = bufB if s % 2 == 0 else bufA
                    nsem = semB if s % 2 == 0 else semA
                    cps[s + 1] = pltpu.async_copy(
                        raw_h.at[slots.at[s + 1]], nxt, nsem)
                cps[s].wait()
                accumulate(buf)
            pltpu.sync_copy(acc, total_out.at[pl.ds(row0, CH)])

        fire_idx_gathers(nbv, curall)

        process(nbv, self_b, total_b, base)

        def nbody(j, carry):
            process(curall.at[j], self_n, total_n, j * B + base)
            return carry
        lax.fori_loop(0, S, nbody, 0)

    return k(raw, nidxT, nodes)


def _tc_fused(sb, tb, sn, tn, wsa, wsb, w2a, w2b):
    f32 = jnp.float32

    def body(sb_r, tb_r, sn_r, tn_r, wsa_r, wsb_r, w2a_r, w2b_r, out_r,
             h1b_s, acc_s):
        j = pl.program_id(1)
        h1n = jnp.maximum(
            jnp.dot(sn_r[:], wsa_r[:], preferred_element_type=f32)
            + jnp.dot(tn_r[:], wsb_r[:], preferred_element_type=f32), 0.0)

        @pl.when(j == 0)
        def _():
            h1b_s[:] = jnp.maximum(
                jnp.dot(sb_r[:], wsa_r[:], preferred_element_type=f32)
                + jnp.dot(tb_r[:], wsb_r[:], preferred_element_type=f32), 0.0)
            acc_s[:] = h1n

        @pl.when(j > 0)
        def _():
            acc_s[:] = acc_s[:] + h1n

        @pl.when(j == S - 1)
        def _():
            h1b = h1b_s[:]
            out_r[:] = jnp.maximum(
                jnp.dot(h1b, w2a_r[:], preferred_element_type=f32)
                + jnp.dot(acc_s[:] + h1b, w2b_r[:], preferred_element_type=f32),
                0.0)

    return pl.pallas_call(
        body,
        grid=(NBLK, S),
        in_specs=[
            pl.BlockSpec((BB, D), lambda ib, j: (ib, 0)),
            pl.BlockSpec((BB, D), lambda ib, j: (ib, 0)),
            pl.BlockSpec((BB, D), lambda ib, j: (j * NBLK + ib, 0)),
            pl.BlockSpec((BB, D), lambda ib, j: (j * NBLK + ib, 0)),
            pl.BlockSpec((D, OUT), lambda ib, j: (0, 0)),
            pl.BlockSpec((D, OUT), lambda ib, j: (0, 0)),
            pl.BlockSpec((OUT, OUT), lambda ib, j: (0, 0)),
            pl.BlockSpec((OUT, OUT), lambda ib, j: (0, 0)),
        ],
        out_specs=pl.BlockSpec((BB, OUT), lambda ib, j: (ib, 0)),
        out_shape=jax.ShapeDtypeStruct((B, OUT), jnp.float32),
        scratch_shapes=[pltpu.VMEM((BB, OUT), jnp.float32),
                        pltpu.VMEM((BB, OUT), jnp.float32)],
        compiler_params=pltpu.CompilerParams(
            dimension_semantics=("arbitrary", "arbitrary")),
    )(sb, tb, sn, tn, wsa, wsb, w2a, w2b)


def kernel(raw_features, neigh_idx, nodes_batch, W1, W2):
    nidxT = neigh_idx.astype(jnp.int32).T.reshape(-1)
    nodes = nodes_batch.astype(jnp.int32)

    self_b, total_b, self_n, total_n = _sc_gather(raw_features, nidxT, nodes)

    inv = 1.0 / (S + 1)
    wsa = W1[:, :D].T
    wsb = W1[:, D:].T * inv
    w2a = W2[:, :OUT].T
    w2b = W2[:, OUT:].T * inv
    return _tc_fused(self_b, total_b, self_n, total_n, wsa, wsb, w2a, w2b)
